# Initial kernel scaffold; baseline (speedup 1.0000x reference)
import functools

import jax
import jax.numpy as jnp
from jax import lax
from jax.experimental import pallas as pl
from jax.experimental.pallas import tpu as pltpu
from jax.experimental.pallas import tpu_sc as plsc

_NC, _NS = 2, 16
_NW = _NC * _NS


def _silu(x):
    return x * (1.0 / (1.0 + jnp.exp(-x)))


def _sc_gather(table, gidx):
    n, d = table.shape
    nw, gn, gc = gidx.shape
    assert nw == _NW and gn % 10 == 0
    nbuf, lag = 10, 5
    out_rows = nw * gn * gc
    mesh = plsc.VectorSubcoreMesh(core_axis_name="c", subcore_axis_name="s")

    @functools.partial(
        pl.kernel,
        out_type=jax.ShapeDtypeStruct((out_rows, d), jnp.float32),
        mesh=mesh,
        scratch_types=[
            pltpu.VMEM((gn, gc), jnp.int32),
            pltpu.VMEM((nbuf, gc, d), jnp.float32),
        ] + [pltpu.SemaphoreType.DMA] * (2 * nbuf),
    )
    def gk(table_h, gidx_h, out_h, idx_v, buf, *sems):
        gsem, wsem = sems[:nbuf], sems[nbuf:]
        c = lax.axis_index("c")
        s = lax.axis_index("s")
        wid = s * _NC + c
        base = wid * (gn * gc)
        pltpu.sync_copy(gidx_h.at[wid], idx_v)

        def gd(j, b):
            return pltpu.make_async_copy(
                table_h.at[idx_v.at[j]], buf.at[b], gsem[b])

        def wd(j, b):
            return pltpu.make_async_copy(
                buf.at[b], out_h.at[pl.ds(base + j * gc, gc)], wsem[b])

        for u in range(lag):
            gd(u, u).start()

        def step(jj, carry):
            j0 = jj * nbuf
            for u in range(nbuf):
                j = j0 + u
                bg = (u + lag) % nbuf

                @pl.when(j >= lag)
                def _():
                    wd(j - lag, bg).wait()

                @pl.when(j + lag < gn)
                def _():
                    gd(j + lag, bg).start()

                gd(j, u).wait()
                wd(j, u).start()
            return carry

        lax.fori_loop(0, gn // nbuf, step, 0)
        for t in range(lag):
            j = gn - lag + t
            wd(j, j % nbuf).wait()

    return gk(table, gidx)


def _sc_scatter(m, dst3, n):
    e_rows, d = m.shape
    nw, sn, sch = dst3.shape
    assert nw == _NW and sn % 10 == 0 and n % _NS == 0
    stripe = n // _NS
    nbuf, lag = 10, 5
    dw = 16
    zagg = jnp.zeros((stripe, d), jnp.float32)
    zdeg = jnp.zeros((stripe, dw), jnp.float32)
    ones = jnp.ones((sch, dw), jnp.float32)
    mesh = plsc.VectorSubcoreMesh(core_axis_name="c", subcore_axis_name="s")

    @functools.partial(
        pl.kernel,
        out_type=(jax.ShapeDtypeStruct((_NC, n, d), jnp.float32),
                  jax.ShapeDtypeStruct((_NC, n, dw), jnp.float32)),
        mesh=mesh,
        scratch_types=[
            pltpu.VMEM((sn, sch), jnp.int32),
            pltpu.VMEM((nbuf, sch, d), jnp.float32),
            pltpu.VMEM((sch, dw), jnp.float32),
            pltpu.VMEM_SHARED((n, d), jnp.float32),
            pltpu.VMEM_SHARED((n, dw), jnp.float32),
        ] + [pltpu.SemaphoreType.DMA] * (3 * nbuf),
    )
    def sk(m_h, dst_h, zagg_h, zdeg_h, ones_h, aggp_h, degp_h,
           idx_v, mbuf, ones_v, agg_sh, deg_sh, *sems):
        lsem = sems[:nbuf]
        ssem = sems[nbuf:2 * nbuf]
        dsem = sems[2 * nbuf:]
        c = lax.axis_index("c")
        s = lax.axis_index("s")
        wid = s * _NC + c
        base = wid * (sn * sch)
        pltpu.sync_copy(zagg_h, agg_sh.at[pl.ds(s * stripe, stripe)])
        pltpu.sync_copy(zdeg_h, deg_sh.at[pl.ds(s * stripe, stripe)])
        pltpu.sync_copy(dst_h.at[wid], idx_v)
        pltpu.sync_copy(ones_h, ones_v)
        plsc.subcore_barrier()

        def ld(j, b):
            return pltpu.make_async_copy(
                m_h.at[pl.ds(base + j * sch, sch)], mbuf.at[b], lsem[b])

        def sd(j, b):
            return pltpu.make_async_copy(
                mbuf.at[b], agg_sh.at[idx_v.at[j]], ssem[b])

        def dd(j, b):
            return pltpu.make_async_copy(
                ones_v, deg_sh.at[idx_v.at[j]], dsem[b])

        for u in range(lag):
            ld(u, u).start()

        def step(jj, carry):
            j0 = jj * nbuf
            for u in range(nbuf):
                j = j0 + u
                bg = (u + lag) % nbuf

                @pl.when(j >= lag)
                def _():
                    sd(j - lag, bg).wait()
                    dd(j - lag, bg).wait()

                @pl.when(j + lag < sn)
                def _():
                    ld(j + lag, bg).start()

                ld(j, u).wait()
                sd(j, u).start(add=True)
                dd(j, u).start(add=True)
            return carry

        lax.fori_loop(0, sn // nbuf, step, 0)
        for t in range(lag):
            j = sn - lag + t
            sd(j, j % nbuf).wait()
            dd(j, j % nbuf).wait()
        plsc.subcore_barrier()
        pltpu.sync_copy(agg_sh.at[pl.ds(s * stripe, stripe)],
                        aggp_h.at[c, pl.ds(s * stripe, stripe)])
        pltpu.sync_copy(deg_sh.at[pl.ds(s * stripe, stripe)],
                        degp_h.at[c, pl.ds(s * stripe, stripe)])

    return sk(m, dst3, zagg, zdeg, ones)


def _mlp(gout, er, ea, ws):
    _, e, d = gout.shape
    comb = er.shape[1] + ea.shape[1]
    be = 1280
    assert e % be == 0

    def body(s_ref, d_ref, er_ref, ea_ref,
             w1s_r, w1d_r, w1e_r, b1_r, w2_r, b2_r, w3_r, b3_r,
             w4_r, b4_r, w5_r, b5_r, w6_r, b6_r,
             nw1d_r, nw1e_r, nb1_r, nw2_r, nb2_r,
             efu_ref, m_ref):
        sf = s_ref[0]
        df = d_ref[0]
        ef = jnp.concatenate([er_ref[...], ea_ref[...]], axis=-1)
        dot = lambda a, b: jnp.dot(a, b, preferred_element_type=jnp.float32)
        h = _silu(dot(sf, w1s_r[...]) + dot(df, w1d_r[...])
                  + dot(ef, w1e_r[...]) + b1_r[...])
        h = _silu(dot(h, w2_r[...]) + b2_r[...])
        h = _silu(dot(h, w3_r[...]) + b3_r[...])
        h4 = dot(h, w4_r[...]) + b4_r[...]
        h = jnp.where(h4 >= 0, h4, 0.01 * h4)
        h = _silu(dot(h, w5_r[...]) + b5_r[...])
        efu = dot(h, w6_r[...]) + b6_r[...] + ef
        efu_ref[...] = efu
        m1 = _silu(dot(df, nw1d_r[...]) + dot(efu, nw1e_r[...]) + nb1_r[...])
        m_ref[...] = dot(m1, nw2_r[...]) + nb2_r[...]

    wspecs = [pl.BlockSpec(w.shape, lambda i, nd=w.ndim: (0,) * nd) for w in ws]
    return pl.pallas_call(
        body,
        grid=(e // be,),
        in_specs=[
            pl.BlockSpec((1, be, d), lambda i: (0, i, 0)),
            pl.BlockSpec((1, be, d), lambda i: (1, i, 0)),
            pl.BlockSpec((be, er.shape[1]), lambda i: (i, 0)),
            pl.BlockSpec((be, ea.shape[1]), lambda i: (i, 0)),
        ] + wspecs,
        out_specs=[
            pl.BlockSpec((be, comb), lambda i: (i, 0)),
            pl.BlockSpec((be, d), lambda i: (i, 0)),
        ],
        out_shape=[
            jax.ShapeDtypeStruct((e, comb), jnp.float32),
            jax.ShapeDtypeStruct((e, d), jnp.float32),
        ],
        compiler_params=pltpu.CompilerParams(
            dimension_semantics=("arbitrary",)),
    )(gout, gout, er, ea, *ws)


def _combine(aggp, degp, nf):
    n, d = nf.shape
    bn = 1000
    assert n % bn == 0
    dw = degp.shape[2]

    def body(a_ref, g_ref, nf_ref, o_ref):
        a = a_ref[0] + a_ref[1]
        dg = g_ref[0, :, 0:1] + g_ref[1, :, 0:1]
        dg = jnp.maximum(dg, 1.0)
        o_ref[...] = a / dg + nf_ref[...]

    return pl.pallas_call(
        body,
        grid=(n // bn,),
        in_specs=[
            pl.BlockSpec((2, bn, d), lambda i: (0, i, 0)),
            pl.BlockSpec((2, bn, dw), lambda i: (0, i, 0)),
            pl.BlockSpec((bn, d), lambda i: (i, 0)),
        ],
        out_specs=pl.BlockSpec((bn, d), lambda i: (i, 0)),
        out_shape=jax.ShapeDtypeStruct((n, d), jnp.float32),
    )(aggp, degp, nf)


def kernel(node_features, edge_radial, edge_angular, edge_index,
           nu_w1, nu_b1, nu_w2, nu_b2,
           eu_w1, eu_b1, eu_w2, eu_b2, eu_w3, eu_b3,
           eu_w4, eu_b4, eu_w5, eu_b5, eu_w6, eu_b6):
    n, d = node_features.shape
    e = edge_index.shape[1]

    gc = 40
    gn = 2 * e // (_NW * gc)
    src = edge_index[0]
    dst = edge_index[1]
    gidx = jnp.concatenate([src, dst]).reshape(_NW, gn, gc)
    gathered = _sc_gather(node_features, gidx)
    gout = gathered.reshape(2, e, d)

    r2 = lambda b: b.reshape(1, -1)
    ws = (eu_w1[:d], eu_w1[d:2 * d], eu_w1[2 * d:], r2(eu_b1),
          eu_w2, r2(eu_b2), eu_w3, r2(eu_b3), eu_w4, r2(eu_b4),
          eu_w5, r2(eu_b5), eu_w6, r2(eu_b6),
          nu_w1[:d], nu_w1[d:], r2(nu_b1), nu_w2, r2(nu_b2))
    efu, m = _mlp(gout, edge_radial, edge_angular, ws)

    sch = 40
    sn = e // (_NW * sch)
    dst3 = dst.reshape(_NW, sn, sch)
    aggp, degp = _sc_scatter(m, dst3, n)

    node_out = _combine(aggp, degp, node_features)
    return node_out, efu


# speedup vs baseline: 1.0432x; 1.0432x over previous
import functools

import jax
import jax.numpy as jnp
from jax import lax
from jax.experimental import pallas as pl
from jax.experimental.pallas import tpu as pltpu
from jax.experimental.pallas import tpu_sc as plsc

_NC, _NS = 2, 16
_NW = _NC * _NS


def _silu(x):
    return x * (1.0 / (1.0 + jnp.exp(-x)))


def _sc_gather(table, gidx):
    n, d = table.shape
    nw, gn, gc = gidx.shape
    assert nw == _NW and gn % 10 == 0
    nbuf, lag = 10, 5
    out_rows = nw * gn * gc
    mesh = plsc.VectorSubcoreMesh(core_axis_name="c", subcore_axis_name="s")

    @functools.partial(
        pl.kernel,
        out_type=jax.ShapeDtypeStruct((out_rows, d), jnp.float32),
        mesh=mesh,
        scratch_types=[
            pltpu.VMEM((gn, gc), jnp.int32),
            pltpu.VMEM((nbuf, gc, d), jnp.float32),
        ] + [pltpu.SemaphoreType.DMA] * (2 * nbuf),
    )
    def gk(table_h, gidx_h, out_h, idx_v, buf, *sems):
        gsem, wsem = sems[:nbuf], sems[nbuf:]
        c = lax.axis_index("c")
        s = lax.axis_index("s")
        wid = s * _NC + c
        base = wid * (gn * gc)
        pltpu.sync_copy(gidx_h.at[wid], idx_v)

        def gd(j, b):
            return pltpu.make_async_copy(
                table_h.at[idx_v.at[j]], buf.at[b], gsem[b])

        def wd(j, b):
            return pltpu.make_async_copy(
                buf.at[b], out_h.at[pl.ds(base + j * gc, gc)], wsem[b])

        for u in range(lag):
            gd(u, u).start()

        def step(jj, carry):
            j0 = jj * nbuf
            for u in range(nbuf):
                j = j0 + u
                bg = (u + lag) % nbuf

                @pl.when(j >= lag)
                def _():
                    wd(j - lag, bg).wait()

                @pl.when(j + lag < gn)
                def _():
                    gd(j + lag, bg).start()

                gd(j, u).wait()
                wd(j, u).start()
            return carry

        lax.fori_loop(0, gn // nbuf, step, 0)
        for t in range(lag):
            j = gn - lag + t
            wd(j, j % nbuf).wait()

    return gk(table, gidx)


def _sc_scatter(m3, dst3, n):
    _, e_rows, dh = m3.shape
    ns, sn, _, sch = dst3.shape
    n = ((n + 8 * _NS - 1) // (8 * _NS)) * (8 * _NS)
    assert ns == _NS and sn % 10 == 0
    stripe = n // _NS
    nbuf, lag = 4, 2
    dw = 16
    zagg = jnp.zeros((128, dh), jnp.float32)
    zdeg = jnp.zeros((128, dw), jnp.float32)
    ones = jnp.ones((sch, dw), jnp.float32)
    mesh = plsc.VectorSubcoreMesh(core_axis_name="c", subcore_axis_name="s")

    @functools.partial(
        pl.kernel,
        out_type=(jax.ShapeDtypeStruct((_NC, n, dh), jnp.float32),
                  jax.ShapeDtypeStruct((n, dw), jnp.float32)),
        mesh=mesh,
        scratch_types=[pltpu.VMEM((sch,), jnp.int32)] * nbuf + [
            pltpu.VMEM((nbuf, sch, dh), jnp.float32),
            pltpu.VMEM((sch, dw), jnp.float32),
            pltpu.VMEM((128, dh), jnp.float32),
            pltpu.VMEM((128, dw), jnp.float32),
            pltpu.VMEM_SHARED((n, dh), jnp.float32),
            pltpu.VMEM_SHARED((n, dw), jnp.float32),
        ] + [pltpu.SemaphoreType.DMA] * (2 * nbuf),
        compiler_params=pltpu.CompilerParams(use_tc_tiling_on_sc=False),
    )
    def sk(m_h, dst_h, zagg_h, zdeg_h, ones_h, aggp_h, degp_h, *rest):
        ibuf = rest[:nbuf]
        mbuf, ones_v, stage, dstage, agg_sh, deg_sh = rest[nbuf:nbuf + 6]
        sems = rest[nbuf + 6:]
        lsem = sems[:nbuf]
        isem = sems[nbuf:2 * nbuf]
        c = lax.axis_index("c")
        s = lax.axis_index("s")
        base = s * (sn * sch)
        chunks = [(o, min(128, stripe - o)) for o in range(0, stripe, 128)]
        pltpu.sync_copy(zagg_h, stage)
        pltpu.sync_copy(zdeg_h, dstage)
        for o, w in chunks:
            pltpu.sync_copy(stage.at[pl.ds(0, w)],
                            agg_sh.at[pl.ds(s * stripe + o, w)])

        @pl.when(c == 0)
        def _():
            for o, w in chunks:
                pltpu.sync_copy(dstage.at[pl.ds(0, w)],
                                deg_sh.at[pl.ds(s * stripe + o, w)])
            pltpu.sync_copy(ones_h, ones_v)

        plsc.subcore_barrier()

        def ld(j, b):
            return pltpu.make_async_copy(
                m_h.at[c, pl.ds(base + j * sch, sch)], mbuf.at[b], lsem[b])

        def li(j, b):
            return pltpu.make_async_copy(dst_h.at[s, j, 0], ibuf[b], isem[b])

        for u in range(nbuf):
            ld(u, u).start()
            li(u, u).start()

        def step(jj, carry):
            j0 = jj * nbuf
            for u in range(nbuf):
                j = j0 + u
                ld(j, u).wait()
                li(j, u).wait()
                pltpu.sync_copy(mbuf.at[u], agg_sh.at[ibuf[u]], add=True)

                @pl.when(c == 0)
                def _():
                    pltpu.sync_copy(ones_v, deg_sh.at[ibuf[u]], add=True)

                @pl.when(j + nbuf < sn)
                def _():
                    ld(j + nbuf, u).start()
                    li(j + nbuf, u).start()
            return carry

        lax.fori_loop(0, sn // nbuf, step, 0)
        plsc.subcore_barrier()
        for o, w in chunks:
            pltpu.sync_copy(agg_sh.at[pl.ds(s * stripe + o, w)],
                            stage.at[pl.ds(0, w)])
            pltpu.sync_copy(stage.at[pl.ds(0, w)],
                            aggp_h.at[c, pl.ds(s * stripe + o, w)])

        @pl.when(c == 0)
        def _():
            for o, w in chunks:
                pltpu.sync_copy(deg_sh.at[pl.ds(s * stripe + o, w)],
                                dstage.at[pl.ds(0, w)])
                pltpu.sync_copy(dstage.at[pl.ds(0, w)],
                                degp_h.at[pl.ds(s * stripe + o, w)])

    return sk(m3, dst3, zagg, zdeg, ones)


def _mlp(gout, er, ea, ws):
    _, e, d = gout.shape
    comb = er.shape[1] + ea.shape[1]
    be = 1280
    assert e % be == 0

    def body(s_ref, d_ref, er_ref, ea_ref,
             w1s_r, w1d_r, w1e_r, b1_r, w2_r, b2_r, w3_r, b3_r,
             w4_r, b4_r, w5_r, b5_r, w6_r, b6_r,
             nw1d_r, nw1e_r, nb1_r, nw2_r, nb2_r,
             efu_ref, m_ref):
        sf = s_ref[0]
        df = d_ref[0]
        ef = jnp.concatenate([er_ref[...], ea_ref[...]], axis=-1)
        dot = lambda a, b: jnp.dot(a, b, preferred_element_type=jnp.float32,
                                   precision=lax.Precision.HIGHEST)
        h = _silu(dot(sf, w1s_r[...]) + dot(df, w1d_r[...])
                  + dot(ef, w1e_r[...]) + b1_r[...])
        h = _silu(dot(h, w2_r[...]) + b2_r[...])
        h = _silu(dot(h, w3_r[...]) + b3_r[...])
        h4 = dot(h, w4_r[...]) + b4_r[...]
        h = jnp.where(h4 >= 0, h4, 0.01 * h4)
        h = _silu(dot(h, w5_r[...]) + b5_r[...])
        efu = dot(h, w6_r[...]) + b6_r[...] + ef
        efu_ref[...] = efu
        m1 = _silu(dot(df, nw1d_r[...]) + dot(efu, nw1e_r[...]) + nb1_r[...])
        mm = dot(m1, nw2_r[...]) + nb2_r[...]
        dh = mm.shape[1] // 2
        m_ref[0, :, :] = mm[:, :dh]
        m_ref[1, :, :] = mm[:, dh:]

    wspecs = [pl.BlockSpec(w.shape, lambda i, nd=w.ndim: (0,) * nd) for w in ws]
    return pl.pallas_call(
        body,
        grid=(e // be,),
        in_specs=[
            pl.BlockSpec((1, be, d), lambda i: (0, i, 0)),
            pl.BlockSpec((1, be, d), lambda i: (1, i, 0)),
            pl.BlockSpec((be, er.shape[1]), lambda i: (i, 0)),
            pl.BlockSpec((be, ea.shape[1]), lambda i: (i, 0)),
        ] + wspecs,
        out_specs=[
            pl.BlockSpec((be, comb), lambda i: (i, 0)),
            pl.BlockSpec((2, be, d // 2), lambda i: (0, i, 0)),
        ],
        out_shape=[
            jax.ShapeDtypeStruct((e, comb), jnp.float32),
            jax.ShapeDtypeStruct((2, e, d // 2), jnp.float32),
        ],
        compiler_params=pltpu.CompilerParams(
            dimension_semantics=("arbitrary",)),
    )(gout, gout, er, ea, *ws)


def _combine(aggp, degp, nf):
    n, d = nf.shape
    bn = 1000
    assert n % bn == 0
    dw = degp.shape[1]

    def body(a_ref, g_ref, nf_ref, o_ref):
        a = jnp.concatenate([a_ref[0], a_ref[1]], axis=-1)
        dg = jnp.maximum(g_ref[:, 0:1], 1.0)
        o_ref[...] = a / dg + nf_ref[...]

    return pl.pallas_call(
        body,
        grid=(n // bn,),
        in_specs=[
            pl.BlockSpec((2, bn, d // 2), lambda i: (0, i, 0)),
            pl.BlockSpec((bn, dw), lambda i: (i, 0)),
            pl.BlockSpec((bn, d), lambda i: (i, 0)),
        ],
        out_specs=pl.BlockSpec((bn, d), lambda i: (i, 0)),
        out_shape=jax.ShapeDtypeStruct((n, d), jnp.float32),
    )(aggp, degp, nf)


def kernel(node_features, edge_radial, edge_angular, edge_index,
           nu_w1, nu_b1, nu_w2, nu_b2,
           eu_w1, eu_b1, eu_w2, eu_b2, eu_w3, eu_b3,
           eu_w4, eu_b4, eu_w5, eu_b5, eu_w6, eu_b6):
    n, d = node_features.shape
    e = edge_index.shape[1]

    gc = 40
    gn = 2 * e // (_NW * gc)
    src = edge_index[0]
    dst = edge_index[1]
    gidx = jnp.concatenate([src, dst]).reshape(_NW, gn, gc)
    gathered = _sc_gather(node_features, gidx)
    gout = gathered.reshape(2, e, d)

    r2 = lambda b: b.reshape(1, -1)
    ws = (eu_w1[:d], eu_w1[d:2 * d], eu_w1[2 * d:], r2(eu_b1),
          eu_w2, r2(eu_b2), eu_w3, r2(eu_b3), eu_w4, r2(eu_b4),
          eu_w5, r2(eu_b5), eu_w6, r2(eu_b6),
          nu_w1[:d], nu_w1[d:], r2(nu_b1), nu_w2, r2(nu_b2))
    efu, m = _mlp(gout, edge_radial, edge_angular, ws)

    sch = 40
    sn = e // (_NS * sch)
    dst3 = dst.reshape(_NS, sn, 1, sch)
    aggp, degp = _sc_scatter(m, dst3, n)

    node_out = _combine(aggp, degp, node_features)
    return node_out, efu
